# Initial kernel scaffold; baseline (speedup 1.0000x reference)
import jax
import jax.numpy as jnp
from jax.experimental import pallas as pl


def kernel(images, ref_masks, w_kernel):
    raise NotImplementedError("write your pallas kernel here")


# speedup vs baseline: 4.3216x; 4.3216x over previous
import jax
import jax.numpy as jnp
from jax.experimental import pallas as pl
from jax.experimental.pallas import tpu as pltpu

_PATCH = 8


def _extract_body(x_ref, m_ref, wt_ref, s4_ref, s2_ref, oi_ref, om_ref):
    n_lat, n_ch = wt_ref.shape[0], wt_ref.shape[1]
    H, W = x_ref.shape[2], x_ref.shape[3]
    gpp = m_ref.shape[1]
    qh = _PATCH // gpp
    hs = H // _PATCH

    s4 = s4_ref[...]
    s2 = s2_ref[...]

    for n in range(n_lat):
        y3 = None
        for c in range(n_ch):
            t = x_ref[0, c].reshape(hs, _PATCH, W) * wt_ref[n, c][None, :, :]
            y3 = t if y3 is None else y3 + t
        acc_i = None
        acc_m = None
        for g in range(gpp):
            part = jnp.sum(y3[:, g * qh:(g + 1) * qh, :], axis=1)
            sg = jnp.dot(part, s4, preferred_element_type=jnp.float32)
            mg = m_ref[0, g] * sg
            acc_i = sg if acc_i is None else acc_i + sg
            acc_m = mg if acc_m is None else acc_m + mg
        oi_ref[0, n] = jnp.dot(acc_i, s2,
                               preferred_element_type=jnp.float32).astype(oi_ref.dtype)
        om_ref[0, n] = jnp.dot(acc_m, s2,
                               preferred_element_type=jnp.float32).astype(om_ref.dtype)


def kernel(images, ref_masks, w_kernel):
    B, C, H, W = images.shape
    _, Hm, Wm = ref_masks.shape
    N = w_kernel.shape[0]
    Hl, Wl = H // _PATCH, W // _PATCH
    qh, qw = H // Hm, W // Wm
    gpp = _PATCH // qh
    cpw = _PATCH // qw
    dt = images.dtype

    wt = jnp.tile(w_kernel.reshape(N, C, _PATCH, _PATCH).astype(dt),
                  (1, 1, 1, W // _PATCH))

    m_parts = ref_masks.astype(dt).reshape(B, Hm // gpp, gpp, Wm)
    m_parts = m_parts.transpose(0, 2, 1, 3)

    s4 = (jnp.arange(W)[:, None] // qw == jnp.arange(Wm)[None, :]).astype(dt)
    s2 = (jnp.arange(Wm)[:, None] // cpw == jnp.arange(Wl)[None, :]).astype(dt)

    out_shape = (jax.ShapeDtypeStruct((B, N, Hl, Wl), dt),
                 jax.ShapeDtypeStruct((B, N, Hl, Wl), dt))
    li, lm = pl.pallas_call(
        _extract_body,
        grid=(B,),
        in_specs=[
            pl.BlockSpec((1, C, H, W), lambda b: (b, 0, 0, 0)),
            pl.BlockSpec((1, gpp, Hm // gpp, Wm), lambda b: (b, 0, 0, 0)),
            pl.BlockSpec((N, C, _PATCH, W), lambda b: (0, 0, 0, 0)),
            pl.BlockSpec((W, Wm), lambda b: (0, 0)),
            pl.BlockSpec((Wm, Wl), lambda b: (0, 0)),
        ],
        out_specs=(pl.BlockSpec((1, N, Hl, Wl), lambda b: (b, 0, 0, 0)),
                   pl.BlockSpec((1, N, Hl, Wl), lambda b: (b, 0, 0, 0))),
        out_shape=out_shape,
        compiler_params=pltpu.CompilerParams(dimension_semantics=("parallel",)),
    )(images, m_parts, wt, s4, s2)
    return li, lm


# speedup vs baseline: 6.0548x; 1.4011x over previous
import jax
import jax.numpy as jnp
from jax.experimental import pallas as pl
from jax.experimental.pallas import tpu as pltpu

_PATCH = 8


def _extract_body(x_ref, m4_ref, wt_ref, s4_ref, rp8_ref, s2_ref,
                  oi_ref, om_ref):
    n_lat, n_ch = wt_ref.shape[0], wt_ref.shape[1]
    H, W = x_ref.shape[2], x_ref.shape[3]
    hs = H // _PATCH

    xb = [x_ref[0, c].astype(jnp.bfloat16).reshape(hs, _PATCH, W)
          for c in range(n_ch)]
    m4 = m4_ref[0]
    s4 = s4_ref[...]
    rp8 = rp8_ref[...]
    s2 = s2_ref[...]

    for n in range(n_lat):
        y = xb[0] * wt_ref[n, 0][None, :, :]
        for c in range(1, n_ch):
            y = y + xb[c] * wt_ref[n, c][None, :, :]
        z = jnp.dot(y.reshape(H, W), s4,
                    preferred_element_type=jnp.float32)
        zi = jnp.dot(rp8, z, preferred_element_type=jnp.float32)
        oi_ref[0, n] = jnp.dot(zi, s2,
                               preferred_element_type=jnp.float32
                               ).astype(oi_ref.dtype)
        zm = jnp.dot(rp8, z * m4, preferred_element_type=jnp.float32)
        om_ref[0, n] = jnp.dot(zm, s2,
                               preferred_element_type=jnp.float32
                               ).astype(om_ref.dtype)


def kernel(images, ref_masks, w_kernel):
    B, C, H, W = images.shape
    _, Hm, Wm = ref_masks.shape
    N = w_kernel.shape[0]
    Hl, Wl = H // _PATCH, W // _PATCH
    qh, qw = H // Hm, W // Wm
    cpw = _PATCH // qw
    dt = images.dtype

    wt = jnp.tile(w_kernel.reshape(N, C, _PATCH, _PATCH),
                  (1, 1, 1, W // _PATCH)).astype(jnp.bfloat16)

    m4 = jnp.repeat(ref_masks.astype(dt), qh, axis=1)

    s4 = (jnp.arange(W)[:, None] // qw == jnp.arange(Wm)[None, :]
          ).astype(jnp.bfloat16)
    rp8 = (jnp.arange(H)[None, :] // _PATCH == jnp.arange(Hl)[:, None]
           ).astype(dt)
    s2 = (jnp.arange(Wm)[:, None] // cpw == jnp.arange(Wl)[None, :]
          ).astype(dt)

    out_shape = (jax.ShapeDtypeStruct((B, N, Hl, Wl), dt),
                 jax.ShapeDtypeStruct((B, N, Hl, Wl), dt))
    li, lm = pl.pallas_call(
        _extract_body,
        grid=(B,),
        in_specs=[
            pl.BlockSpec((1, C, H, W), lambda b: (b, 0, 0, 0)),
            pl.BlockSpec((1, H, Wm), lambda b: (b, 0, 0)),
            pl.BlockSpec((N, C, _PATCH, W), lambda b: (0, 0, 0, 0)),
            pl.BlockSpec((W, Wm), lambda b: (0, 0)),
            pl.BlockSpec((Hl, H), lambda b: (0, 0)),
            pl.BlockSpec((Wm, Wl), lambda b: (0, 0)),
        ],
        out_specs=(pl.BlockSpec((1, N, Hl, Wl), lambda b: (b, 0, 0, 0)),
                   pl.BlockSpec((1, N, Hl, Wl), lambda b: (b, 0, 0, 0))),
        out_shape=out_shape,
        compiler_params=pltpu.CompilerParams(dimension_semantics=("parallel",)),
    )(images, m4, wt, s4, rp8, s2)
    return li, lm


# speedup vs baseline: 8.4594x; 1.3971x over previous
import jax
import jax.numpy as jnp
from jax.experimental import pallas as pl
from jax.experimental.pallas import tpu as pltpu

_PATCH = 8


def _extract_body(x_ref, m4_ref, wt_ref, s4_ref, rp8_ref, s2_ref,
                  oi_ref, om_ref):
    n_lat, n_ch = wt_ref.shape[0], wt_ref.shape[1]
    H, W = x_ref.shape[2], x_ref.shape[3]
    Wm = s4_ref.shape[1]
    Hl = rp8_ref.shape[0]
    hs = H // _PATCH

    xb = [x_ref[0, c].astype(jnp.bfloat16).reshape(hs, _PATCH, W)
          for c in range(n_ch)]
    m4 = m4_ref[0]

    ys = []
    for n in range(n_lat):
        y = xb[0] * wt_ref[n, 0][None, :, :]
        for c in range(1, n_ch):
            y = y + xb[c] * wt_ref[n, c][None, :, :]
        ys.append(y.reshape(H, W))
    yall = jnp.concatenate(ys, axis=0)

    z = jnp.dot(yall, s4_ref[...],
                preferred_element_type=jnp.float32)
    zb = z.astype(jnp.bfloat16)

    pieces = []
    for n in range(n_lat):
        zn = zb[n * H:(n + 1) * H]
        pieces.append(zn)
        pieces.append(zn * m4)
    zall = jnp.concatenate(pieces, axis=1)
    t = jnp.dot(rp8_ref[...], zall,
                preferred_element_type=jnp.float32)

    s = jnp.concatenate(
        [t[:, k * Wm:(k + 1) * Wm] for k in range(2 * n_lat)], axis=0)
    o = jnp.dot(s, s2_ref[...],
                preferred_element_type=jnp.float32)

    for n in range(n_lat):
        oi_ref[0, n] = o[2 * n * Hl:(2 * n + 1) * Hl].astype(oi_ref.dtype)
        om_ref[0, n] = o[(2 * n + 1) * Hl:(2 * n + 2) * Hl].astype(om_ref.dtype)


def kernel(images, ref_masks, w_kernel):
    B, C, H, W = images.shape
    _, Hm, Wm = ref_masks.shape
    N = w_kernel.shape[0]
    Hl, Wl = H // _PATCH, W // _PATCH
    qh, qw = H // Hm, W // Wm
    cpw = _PATCH // qw
    dt = images.dtype

    wt = jnp.tile(w_kernel.reshape(N, C, _PATCH, _PATCH),
                  (1, 1, 1, W // _PATCH)).astype(jnp.bfloat16)

    m4 = jnp.repeat(ref_masks, qh, axis=1).astype(jnp.bfloat16)

    s4 = (jnp.arange(W)[:, None] // qw == jnp.arange(Wm)[None, :]
          ).astype(jnp.bfloat16)
    rp8 = (jnp.arange(H)[None, :] // _PATCH == jnp.arange(Hl)[:, None]
           ).astype(jnp.bfloat16)
    s2 = (jnp.arange(Wm)[:, None] // cpw == jnp.arange(Wl)[None, :]
          ).astype(dt)

    out_shape = (jax.ShapeDtypeStruct((B, N, Hl, Wl), dt),
                 jax.ShapeDtypeStruct((B, N, Hl, Wl), dt))
    li, lm = pl.pallas_call(
        _extract_body,
        grid=(B,),
        in_specs=[
            pl.BlockSpec((1, C, H, W), lambda b: (b, 0, 0, 0)),
            pl.BlockSpec((1, H, Wm), lambda b: (b, 0, 0)),
            pl.BlockSpec((N, C, _PATCH, W), lambda b: (0, 0, 0, 0)),
            pl.BlockSpec((W, Wm), lambda b: (0, 0)),
            pl.BlockSpec((Hl, H), lambda b: (0, 0)),
            pl.BlockSpec((Wm, Wl), lambda b: (0, 0)),
        ],
        out_specs=(pl.BlockSpec((1, N, Hl, Wl), lambda b: (b, 0, 0, 0)),
                   pl.BlockSpec((1, N, Hl, Wl), lambda b: (b, 0, 0, 0))),
        out_shape=out_shape,
        compiler_params=pltpu.CompilerParams(dimension_semantics=("parallel",)),
    )(images, m4, wt, s4, rp8, s2)
    return li, lm


# speedup vs baseline: 8.8597x; 1.0473x over previous
import jax
import jax.numpy as jnp
from jax.experimental import pallas as pl
from jax.experimental.pallas import tpu as pltpu

_PATCH = 8


def _extract_body(x_ref, m_ref, wt_ref, s4_ref, rp4_ref, rp2_ref, s2_ref,
                  oi_ref, om_ref):
    n_lat, n_ch = wt_ref.shape[0], wt_ref.shape[1]
    H, W = x_ref.shape[2], x_ref.shape[3]
    Wm = s4_ref.shape[1]
    Hl = rp2_ref.shape[0]
    hs = H // _PATCH

    xb = [x_ref[0, c].astype(jnp.bfloat16).reshape(hs, _PATCH, W)
          for c in range(n_ch)]
    mb = m_ref[0].astype(jnp.bfloat16)

    ys = []
    for n in range(n_lat):
        y = xb[0] * wt_ref[n, 0][None, :, :]
        for c in range(1, n_ch):
            y = y + xb[c] * wt_ref[n, c][None, :, :]
        ys.append(y.reshape(H, W))
    yall = jnp.concatenate(ys, axis=0)

    z = jnp.dot(yall, s4_ref[...],
                preferred_element_type=jnp.float32)
    zb = z.astype(jnp.bfloat16)

    zlanes = jnp.concatenate(
        [zb[n * H:(n + 1) * H] for n in range(n_lat)], axis=1)
    t4 = jnp.dot(rp4_ref[...], zlanes,
                 preferred_element_type=jnp.float32)
    t4b = t4.astype(jnp.bfloat16)

    pieces = []
    for n in range(n_lat):
        tn = t4b[:, n * Wm:(n + 1) * Wm]
        pieces.append(tn)
        pieces.append(tn * mb)
    tall = jnp.concatenate(pieces, axis=1)
    t2 = jnp.dot(rp2_ref[...], tall,
                 preferred_element_type=jnp.float32)

    s = jnp.concatenate(
        [t2[:, k * Wm:(k + 1) * Wm] for k in range(2 * n_lat)], axis=0)
    o = jnp.dot(s, s2_ref[...],
                preferred_element_type=jnp.float32)

    for n in range(n_lat):
        oi_ref[0, n] = o[2 * n * Hl:(2 * n + 1) * Hl].astype(oi_ref.dtype)
        om_ref[0, n] = o[(2 * n + 1) * Hl:(2 * n + 2) * Hl].astype(om_ref.dtype)


def kernel(images, ref_masks, w_kernel):
    B, C, H, W = images.shape
    _, Hm, Wm = ref_masks.shape
    N = w_kernel.shape[0]
    Hl, Wl = H // _PATCH, W // _PATCH
    qh, qw = H // Hm, W // Wm
    cph = _PATCH // qh
    cpw = _PATCH // qw
    dt = images.dtype

    wt = jnp.tile(w_kernel.reshape(N, C, _PATCH, _PATCH),
                  (1, 1, 1, W // _PATCH)).astype(jnp.bfloat16)

    s4 = (jnp.arange(W)[:, None] // qw == jnp.arange(Wm)[None, :]
          ).astype(jnp.bfloat16)
    rp4 = (jnp.arange(H)[None, :] // qh == jnp.arange(Hm)[:, None]
           ).astype(jnp.bfloat16)
    rp2 = (jnp.arange(Hm)[None, :] // cph == jnp.arange(Hl)[:, None]
           ).astype(jnp.bfloat16)
    s2 = (jnp.arange(Wm)[:, None] // cpw == jnp.arange(Wl)[None, :]
          ).astype(dt)

    out_shape = (jax.ShapeDtypeStruct((B, N, Hl, Wl), dt),
                 jax.ShapeDtypeStruct((B, N, Hl, Wl), dt))
    li, lm = pl.pallas_call(
        _extract_body,
        grid=(B,),
        in_specs=[
            pl.BlockSpec((1, C, H, W), lambda b: (b, 0, 0, 0)),
            pl.BlockSpec((1, Hm, Wm), lambda b: (b, 0, 0)),
            pl.BlockSpec((N, C, _PATCH, W), lambda b: (0, 0, 0, 0)),
            pl.BlockSpec((W, Wm), lambda b: (0, 0)),
            pl.BlockSpec((Hm, H), lambda b: (0, 0)),
            pl.BlockSpec((Hl, Hm), lambda b: (0, 0)),
            pl.BlockSpec((Wm, Wl), lambda b: (0, 0)),
        ],
        out_specs=(pl.BlockSpec((1, N, Hl, Wl), lambda b: (b, 0, 0, 0)),
                   pl.BlockSpec((1, N, Hl, Wl), lambda b: (b, 0, 0, 0))),
        out_shape=out_shape,
        compiler_params=pltpu.CompilerParams(dimension_semantics=("parallel",)),
    )(images, ref_masks, wt, s4, rp4, rp2, s2)
    return li, lm


# speedup vs baseline: 9.7529x; 1.1008x over previous
import numpy as np
import jax
import jax.numpy as jnp
from jax.experimental import pallas as pl
from jax.experimental.pallas import tpu as pltpu

_PATCH = 8
_IMGS_PER_STEP = 2


def _extract_body(x_ref, m_ref, wt_ref, s4_ref, rp4_ref, rp2_ref, s2_ref,
                  oi_ref, om_ref):
    n_lat, n_ch = wt_ref.shape[0], wt_ref.shape[1]
    H, W = x_ref.shape[2], x_ref.shape[3]
    Wm = s4_ref.shape[1]
    Hl = rp2_ref.shape[0]
    hs = H // _PATCH

    for g in range(x_ref.shape[0]):
        xb = [x_ref[g, c].astype(jnp.bfloat16).reshape(hs, _PATCH, W)
              for c in range(n_ch)]
        mb = m_ref[g].astype(jnp.bfloat16)

        ys = []
        for n in range(n_lat):
            y = xb[0] * wt_ref[n, 0][None, :, :]
            for c in range(1, n_ch):
                y = y + xb[c] * wt_ref[n, c][None, :, :]
            ys.append(y.reshape(H, W))
        yall = jnp.concatenate(ys, axis=0)

        z = jnp.dot(yall, s4_ref[...],
                    preferred_element_type=jnp.float32)
        zb = z.astype(jnp.bfloat16)

        zlanes = jnp.concatenate(
            [zb[n * H:(n + 1) * H] for n in range(n_lat)], axis=1)
        t4 = jnp.dot(rp4_ref[...], zlanes,
                     preferred_element_type=jnp.float32)
        t4b = t4.astype(jnp.bfloat16)

        pieces = []
        for n in range(n_lat):
            tn = t4b[:, n * Wm:(n + 1) * Wm]
            pieces.append(tn)
            pieces.append(tn * mb)
        tall = jnp.concatenate(pieces, axis=1)
        t2 = jnp.dot(rp2_ref[...], tall,
                     preferred_element_type=jnp.float32)

        s = jnp.concatenate(
            [t2[:, k * Wm:(k + 1) * Wm] for k in range(2 * n_lat)], axis=0)
        o = jnp.dot(s, s2_ref[...],
                    preferred_element_type=jnp.float32)

        for n in range(n_lat):
            oi_ref[g, n] = o[2 * n * Hl:(2 * n + 1) * Hl].astype(oi_ref.dtype)
            om_ref[g, n] = o[(2 * n + 1) * Hl:
                             (2 * n + 2) * Hl].astype(om_ref.dtype)


def kernel(images, ref_masks, w_kernel):
    B, C, H, W = images.shape
    _, Hm, Wm = ref_masks.shape
    N = w_kernel.shape[0]
    Hl, Wl = H // _PATCH, W // _PATCH
    qh, qw = H // Hm, W // Wm
    cph = _PATCH // qh
    cpw = _PATCH // qw
    dt = images.dtype
    bf = jnp.bfloat16

    t8 = np.equal(np.arange(W)[None, :] % _PATCH,
                  np.arange(_PATCH)[:, None]).astype(np.float32)
    wt = jnp.einsum('ncjd,dw->ncjw',
                    w_kernel.reshape(N, C, _PATCH, _PATCH), t8,
                    precision=jax.lax.Precision.HIGHEST).astype(bf)

    s4 = jnp.asarray(np.equal(np.arange(W)[:, None] // qw,
                              np.arange(Wm)[None, :]), dtype=bf)
    rp4 = jnp.asarray(np.equal(np.arange(H)[None, :] // qh,
                               np.arange(Hm)[:, None]), dtype=bf)
    rp2 = jnp.asarray(np.equal(np.arange(Hm)[None, :] // cph,
                               np.arange(Hl)[:, None]), dtype=bf)
    s2 = jnp.asarray(np.equal(np.arange(Wm)[:, None] // cpw,
                              np.arange(Wl)[None, :]),
                     dtype=np.dtype(dt.name) if hasattr(dt, 'name') else dt)

    G = _IMGS_PER_STEP if B % _IMGS_PER_STEP == 0 else 1
    out_shape = (jax.ShapeDtypeStruct((B, N, Hl, Wl), dt),
                 jax.ShapeDtypeStruct((B, N, Hl, Wl), dt))
    li, lm = pl.pallas_call(
        _extract_body,
        grid=(B // G,),
        in_specs=[
            pl.BlockSpec((G, C, H, W), lambda b: (b, 0, 0, 0)),
            pl.BlockSpec((G, Hm, Wm), lambda b: (b, 0, 0)),
            pl.BlockSpec((N, C, _PATCH, W), lambda b: (0, 0, 0, 0)),
            pl.BlockSpec((W, Wm), lambda b: (0, 0)),
            pl.BlockSpec((Hm, H), lambda b: (0, 0)),
            pl.BlockSpec((Hl, Hm), lambda b: (0, 0)),
            pl.BlockSpec((Wm, Wl), lambda b: (0, 0)),
        ],
        out_specs=(pl.BlockSpec((G, N, Hl, Wl), lambda b: (b, 0, 0, 0)),
                   pl.BlockSpec((G, N, Hl, Wl), lambda b: (b, 0, 0, 0))),
        out_shape=out_shape,
        compiler_params=pltpu.CompilerParams(dimension_semantics=("parallel",)),
    )(images, ref_masks, wt, s4, rp4, rp2, s2)
    return li, lm


# speedup vs baseline: 12.1302x; 1.2437x over previous
import numpy as np
import jax
import jax.numpy as jnp
from jax.experimental import pallas as pl
from jax.experimental.pallas import tpu as pltpu

_PATCH = 8
_IMGS_PER_STEP = 2


def _extract_body(x_ref, m_ref, wt_ref, s4_ref, rp4_ref, rp2_ref, s2_ref,
                  oi_ref, om_ref):
    n_lat, n_ch = wt_ref.shape[0], wt_ref.shape[1]
    H, W = x_ref.shape[2], x_ref.shape[3]
    Wm = s4_ref.shape[1]
    Hm = rp4_ref.shape[0]
    Hl = rp2_ref.shape[0]
    hs = H // _PATCH

    for g in range(x_ref.shape[0]):
        xb = [x_ref[g, c].astype(jnp.bfloat16).reshape(hs, _PATCH, W)
              for c in range(n_ch)]
        mb = m_ref[g].astype(jnp.bfloat16)

        ts = []
        for n in range(n_lat):
            y = xb[0] * wt_ref[n, 0][None, :, :]
            for c in range(1, n_ch):
                y = y + xb[c] * wt_ref[n, c][None, :, :]
            ts.append(jnp.dot(rp4_ref[...], y.reshape(H, W),
                              preferred_element_type=jnp.float32))
        t4 = jnp.concatenate(ts, axis=0).astype(jnp.bfloat16)

        z = jnp.dot(t4, s4_ref[...],
                    preferred_element_type=jnp.float32)
        zb = z.astype(jnp.bfloat16)

        pieces = []
        for n in range(n_lat):
            zn = zb[n * Hm:(n + 1) * Hm]
            pieces.append(zn)
            pieces.append(zn * mb)
        tall = jnp.concatenate(pieces, axis=1)
        t2 = jnp.dot(rp2_ref[...], tall,
                     preferred_element_type=jnp.float32)

        s = jnp.concatenate(
            [t2[:, k * Wm:(k + 1) * Wm] for k in range(2 * n_lat)], axis=0)
        o = jnp.dot(s, s2_ref[...],
                    preferred_element_type=jnp.float32)

        for n in range(n_lat):
            oi_ref[g, n] = o[2 * n * Hl:(2 * n + 1) * Hl].astype(oi_ref.dtype)
            om_ref[g, n] = o[(2 * n + 1) * Hl:
                             (2 * n + 2) * Hl].astype(om_ref.dtype)


def kernel(images, ref_masks, w_kernel):
    B, C, H, W = images.shape
    _, Hm, Wm = ref_masks.shape
    N = w_kernel.shape[0]
    Hl, Wl = H // _PATCH, W // _PATCH
    qh, qw = H // Hm, W // Wm
    cph = _PATCH // qh
    cpw = _PATCH // qw
    dt = images.dtype
    bf = jnp.bfloat16

    t8 = np.equal(np.arange(W)[None, :] % _PATCH,
                  np.arange(_PATCH)[:, None]).astype(np.float32)
    wt = jnp.einsum('ncjd,dw->ncjw',
                    w_kernel.reshape(N, C, _PATCH, _PATCH), t8,
                    precision=jax.lax.Precision.HIGHEST).astype(bf)

    s4 = jnp.asarray(np.equal(np.arange(W)[:, None] // qw,
                              np.arange(Wm)[None, :]), dtype=bf)
    rp4 = jnp.asarray(np.equal(np.arange(H)[None, :] // qh,
                               np.arange(Hm)[:, None]), dtype=bf)
    rp2 = jnp.asarray(np.equal(np.arange(Hm)[None, :] // cph,
                               np.arange(Hl)[:, None]), dtype=bf)
    s2 = jnp.asarray(np.equal(np.arange(Wm)[:, None] // cpw,
                              np.arange(Wl)[None, :]),
                     dtype=np.dtype(dt.name) if hasattr(dt, 'name') else dt)

    G = _IMGS_PER_STEP if B % _IMGS_PER_STEP == 0 else 1
    out_shape = (jax.ShapeDtypeStruct((B, N, Hl, Wl), dt),
                 jax.ShapeDtypeStruct((B, N, Hl, Wl), dt))
    li, lm = pl.pallas_call(
        _extract_body,
        grid=(B // G,),
        in_specs=[
            pl.BlockSpec((G, C, H, W), lambda b: (b, 0, 0, 0)),
            pl.BlockSpec((G, Hm, Wm), lambda b: (b, 0, 0)),
            pl.BlockSpec((N, C, _PATCH, W), lambda b: (0, 0, 0, 0)),
            pl.BlockSpec((W, Wm), lambda b: (0, 0)),
            pl.BlockSpec((Hm, H), lambda b: (0, 0)),
            pl.BlockSpec((Hl, Hm), lambda b: (0, 0)),
            pl.BlockSpec((Wm, Wl), lambda b: (0, 0)),
        ],
        out_specs=(pl.BlockSpec((G, N, Hl, Wl), lambda b: (b, 0, 0, 0)),
                   pl.BlockSpec((G, N, Hl, Wl), lambda b: (b, 0, 0, 0))),
        out_shape=out_shape,
        compiler_params=pltpu.CompilerParams(dimension_semantics=("parallel",)),
    )(images, ref_masks, wt, s4, rp4, rp2, s2)
    return li, lm


# speedup vs baseline: 13.2711x; 1.0941x over previous
import numpy as np
import jax
import jax.numpy as jnp
from jax.experimental import pallas as pl
from jax.experimental.pallas import tpu as pltpu

_PATCH = 8
_IMGS_PER_STEP = 4


def _extract_body(x_ref, m_ref, wt_ref, s4_ref, rp4_ref, rp2_ref, s2_ref,
                  oi_ref, om_ref):
    n_lat, n_ch = wt_ref.shape[0], wt_ref.shape[1]
    H, W = x_ref.shape[2], x_ref.shape[3]
    Wm = s4_ref.shape[1]
    Hm = rp4_ref.shape[0]
    Hl = rp2_ref.shape[0]
    hs = H // _PATCH

    for g in range(x_ref.shape[0]):
        xb = [x_ref[g, c].astype(jnp.bfloat16).reshape(hs, _PATCH, W)
              for c in range(n_ch)]
        mb = m_ref[g].astype(jnp.bfloat16)

        ts = []
        for n in range(n_lat):
            y = xb[0] * wt_ref[n, 0][None, :, :]
            for c in range(1, n_ch):
                y = y + xb[c] * wt_ref[n, c][None, :, :]
            ts.append(jnp.dot(rp4_ref[...], y.reshape(H, W),
                              preferred_element_type=jnp.float32))
        t4 = jnp.concatenate(ts, axis=0).astype(jnp.bfloat16)

        z = jnp.dot(t4, s4_ref[...],
                    preferred_element_type=jnp.float32)
        zb = z.astype(jnp.bfloat16)

        pieces = []
        for n in range(n_lat):
            zn = zb[n * Hm:(n + 1) * Hm]
            pieces.append(zn)
            pieces.append(zn * mb)
        tall = jnp.concatenate(pieces, axis=1)
        t2 = jnp.dot(rp2_ref[...], tall,
                     preferred_element_type=jnp.float32)

        s = jnp.concatenate(
            [t2[:, k * Wm:(k + 1) * Wm] for k in range(2 * n_lat)], axis=0)
        o = jnp.dot(s, s2_ref[...],
                    preferred_element_type=jnp.float32)

        for n in range(n_lat):
            oi_ref[g, n] = o[2 * n * Hl:(2 * n + 1) * Hl].astype(oi_ref.dtype)
            om_ref[g, n] = o[(2 * n + 1) * Hl:
                             (2 * n + 2) * Hl].astype(om_ref.dtype)


def kernel(images, ref_masks, w_kernel):
    B, C, H, W = images.shape
    _, Hm, Wm = ref_masks.shape
    N = w_kernel.shape[0]
    Hl, Wl = H // _PATCH, W // _PATCH
    qh, qw = H // Hm, W // Wm
    cph = _PATCH // qh
    cpw = _PATCH // qw
    dt = images.dtype
    bf = jnp.bfloat16

    t8 = np.equal(np.arange(W)[None, :] % _PATCH,
                  np.arange(_PATCH)[:, None]).astype(np.float32)
    wt = jnp.einsum('ncjd,dw->ncjw',
                    w_kernel.reshape(N, C, _PATCH, _PATCH), t8,
                    precision=jax.lax.Precision.HIGHEST).astype(bf)

    s4 = jnp.asarray(np.equal(np.arange(W)[:, None] // qw,
                              np.arange(Wm)[None, :]), dtype=bf)
    rp4 = jnp.asarray(np.equal(np.arange(H)[None, :] // qh,
                               np.arange(Hm)[:, None]), dtype=bf)
    rp2 = jnp.asarray(np.equal(np.arange(Hm)[None, :] // cph,
                               np.arange(Hl)[:, None]), dtype=bf)
    s2 = jnp.asarray(np.equal(np.arange(Wm)[:, None] // cpw,
                              np.arange(Wl)[None, :]),
                     dtype=np.dtype(dt.name) if hasattr(dt, 'name') else dt)

    G = _IMGS_PER_STEP if B % _IMGS_PER_STEP == 0 else 1
    out_shape = (jax.ShapeDtypeStruct((B, N, Hl, Wl), dt),
                 jax.ShapeDtypeStruct((B, N, Hl, Wl), dt))
    li, lm = pl.pallas_call(
        _extract_body,
        grid=(B // G,),
        in_specs=[
            pl.BlockSpec((G, C, H, W), lambda b: (b, 0, 0, 0)),
            pl.BlockSpec((G, Hm, Wm), lambda b: (b, 0, 0)),
            pl.BlockSpec((N, C, _PATCH, W), lambda b: (0, 0, 0, 0)),
            pl.BlockSpec((W, Wm), lambda b: (0, 0)),
            pl.BlockSpec((Hm, H), lambda b: (0, 0)),
            pl.BlockSpec((Hl, Hm), lambda b: (0, 0)),
            pl.BlockSpec((Wm, Wl), lambda b: (0, 0)),
        ],
        out_specs=(pl.BlockSpec((G, N, Hl, Wl), lambda b: (b, 0, 0, 0)),
                   pl.BlockSpec((G, N, Hl, Wl), lambda b: (b, 0, 0, 0))),
        out_shape=out_shape,
        compiler_params=pltpu.CompilerParams(dimension_semantics=("parallel",)),
    )(images, ref_masks, wt, s4, rp4, rp2, s2)
    return li, lm
